# baseline (speedup 1.0000x reference)
import jax
import jax.numpy as jnp
from jax.experimental import pallas as pl
from jax.experimental.pallas import tpu as pltpu

NUM_MOVIES = 100000
L_OVRV, L_CAST, L_GENRE, L_PC, L_PCO = 200, 50, 5, 5, 3
D_TITLE, D_OVRV, D_DIR, D_CAST, D_GENRE, D_PC, D_PCO, D_NUM = (
    20, 20, 8, 10, 15, 10, 10, 16)


def _body(m_ref, title_h, ovrv_h, dir_h, cast_h, genre_h, pc_h, pco_h, num_h,
          wt_h, wo_h, wd_h, wc_h, wg_h, wp_h, wq_h, out_ref,
          si_o, si_c, si_g, si_p, si_q, si_t, si_d,
          wv_o, wv_c, wv_g, wv_p, wv_q, num_v, rt, rd, sem1, semw):
  m = m_ref[0, 0]

  hw = [
      pltpu.make_async_copy(wo_h, wv_o, semw.at[0]),
      pltpu.make_async_copy(wc_h, wv_c, semw.at[1]),
      pltpu.make_async_copy(wg_h, wv_g, semw.at[2]),
      pltpu.make_async_copy(wp_h, wv_p, semw.at[3]),
      pltpu.make_async_copy(wq_h, wv_q, semw.at[4]),
  ]
  for h in hw:
    h.start()

  h1 = [
      pltpu.make_async_copy(ovrv_h.at[pl.ds(m, 1)], si_o, sem1.at[0]),
      pltpu.make_async_copy(cast_h.at[pl.ds(m, 1)], si_c, sem1.at[1]),
      pltpu.make_async_copy(genre_h.at[pl.ds(m, 1)], si_g, sem1.at[2]),
      pltpu.make_async_copy(pc_h.at[pl.ds(m, 1)], si_p, sem1.at[3]),
      pltpu.make_async_copy(pco_h.at[pl.ds(m, 1)], si_q, sem1.at[4]),
      pltpu.make_async_copy(title_h.at[pl.ds(m, 1)], si_t, sem1.at[5]),
      pltpu.make_async_copy(dir_h.at[pl.ds(m, 1)], si_d, sem1.at[6]),
      pltpu.make_async_copy(num_h.at[pl.ds(m, 1)], num_v, sem1.at[7]),
  ]
  for h in h1:
    h.start()
  for h in h1:
    h.wait()

  ht = pltpu.make_async_copy(wt_h.at[pl.ds(si_t[0, 0], 1)], rt, sem1.at[5])
  hd = pltpu.make_async_copy(wd_h.at[pl.ds(si_d[0, 0], 1)], rd, sem1.at[6])
  ht.start()
  hd.start()
  for h in hw:
    h.wait()

  def pool(row, si, L, wv, D, scale):
    s = wv[pl.ds(si[0, 0], 1), :]
    for j in range(1, L):
      s = s + wv[pl.ds(si[0, j], 1), :]
    out_ref[pl.ds(row, 1), pl.ds(0, D)] = s * jnp.float32(scale)

  pool(1, si_o, L_OVRV, wv_o, D_OVRV, 1.0 / L_OVRV)
  pool(3, si_c, L_CAST, wv_c, D_CAST, 1.0 / L_CAST)
  pool(4, si_g, L_GENRE, wv_g, D_GENRE, 1.0 / L_GENRE)
  pool(5, si_p, L_PC, wv_p, D_PC, 1.0 / L_PC)
  pool(6, si_q, L_PCO, wv_q, D_PCO, 1.0 / L_PCO)
  out_ref[pl.ds(7, 1), pl.ds(0, D_NUM)] = num_v[...]
  ht.wait()
  hd.wait()
  out_ref[pl.ds(0, 1), pl.ds(0, D_TITLE)] = rt[...]
  out_ref[pl.ds(2, 1), pl.ds(0, D_DIR)] = rd[...]


@jax.jit
def _tc_call(m2, title, ovrv, director, cast, genre, pc, pco, num, wt, wo,
             wd, wc, wg, wp, wq):
  out8 = pl.pallas_call(
      _body,
      out_shape=jax.ShapeDtypeStruct((8, 128), jnp.float32),
      in_specs=[pl.BlockSpec(memory_space=pltpu.SMEM)] +
               [pl.BlockSpec(memory_space=pltpu.MemorySpace.HBM)] * 15,
      out_specs=pl.BlockSpec(memory_space=pltpu.VMEM),
      scratch_shapes=[
          pltpu.SMEM((1, L_OVRV), jnp.int32),
          pltpu.SMEM((1, L_CAST), jnp.int32),
          pltpu.SMEM((1, L_GENRE), jnp.int32),
          pltpu.SMEM((1, L_PC), jnp.int32),
          pltpu.SMEM((1, L_PCO), jnp.int32),
          pltpu.SMEM((1, 1), jnp.int32),
          pltpu.SMEM((1, 1), jnp.int32),
          pltpu.VMEM((30000, D_OVRV), jnp.float32),
          pltpu.VMEM((20000, D_CAST), jnp.float32),
          pltpu.VMEM((32, D_GENRE), jnp.float32),
          pltpu.VMEM((10000, D_PC), jnp.float32),
          pltpu.VMEM((200, D_PCO), jnp.float32),
          pltpu.VMEM((1, D_NUM), jnp.float32),
          pltpu.VMEM((1, D_TITLE), jnp.float32),
          pltpu.VMEM((1, D_DIR), jnp.float32),
          pltpu.SemaphoreType.DMA((8,)),
          pltpu.SemaphoreType.DMA((5,)),
      ],
  )(m2, title, ovrv, director, cast, genre, pc, pco, num, wt, wo, wd, wc,
    wg, wp, wq)
  return jnp.concatenate(
      (out8[0, :D_TITLE], out8[1, :D_OVRV], out8[2, :D_DIR],
       out8[3, :D_CAST], out8[4, :D_GENRE], out8[5, :D_PC], out8[6, :D_PCO],
       out8[7, :D_NUM]))


def kernel(movie_ids, title, overrview, director, cast, genre,
           production_compaines, production_countries, numeric_movie_data,
           W_title, W_ovrv, W_dir, W_cast, W_genre, W_pc, W_pco):
  m2 = jnp.reshape(jnp.asarray(movie_ids, jnp.int32) - 1, (1, 1))
  title2 = jnp.reshape(title, (NUM_MOVIES, 1))
  dir2 = jnp.reshape(director, (NUM_MOVIES, 1))
  return _tc_call(m2, title2, overrview, dir2, cast, genre,
                  production_compaines, production_countries,
                  numeric_movie_data, W_title, W_ovrv, W_dir, W_cast,
                  W_genre, W_pc, W_pco)


# speedup vs baseline: 5.4411x; 5.4411x over previous
import jax
import jax.numpy as jnp
from jax.experimental import pallas as pl
from jax.experimental.pallas import tpu as pltpu

NUM_MOVIES = 100000
L_OVRV, L_CAST, L_GENRE, L_PC, L_PCO = 200, 50, 5, 5, 3
D_TITLE, D_OVRV, D_DIR, D_CAST, D_GENRE, D_PC, D_PCO, D_NUM = (
    20, 20, 8, 10, 15, 10, 10, 16)


def _body(si_o, si_c, si_g, si_p, si_q, si_t, si_d, num_in,
          wt_h, wo_h, wd_h, wc_h, wg_h, wp_h, wq_h, out_ref,
          wv_o, wv_c, wv_g, wv_p, wv_q, rt, rd, sem1, semw):
  hw = [
      pltpu.make_async_copy(wo_h, wv_o, semw.at[0]),
      pltpu.make_async_copy(wc_h, wv_c, semw.at[1]),
      pltpu.make_async_copy(wg_h, wv_g, semw.at[2]),
      pltpu.make_async_copy(wp_h, wv_p, semw.at[3]),
      pltpu.make_async_copy(wq_h, wv_q, semw.at[4]),
  ]
  for h in hw:
    h.start()

  ht = pltpu.make_async_copy(wt_h.at[pl.ds(si_t[0, 0], 1)], rt, sem1.at[0])
  hd = pltpu.make_async_copy(wd_h.at[pl.ds(si_d[0, 0], 1)], rd, sem1.at[1])
  ht.start()
  hd.start()
  for h in hw:
    h.wait()

  def pool(row, si, L, wv, D, scale):
    s = wv[pl.ds(si[0, 0], 1), :]
    for j in range(1, L):
      s = s + wv[pl.ds(si[0, j], 1), :]
    out_ref[pl.ds(row, 1), pl.ds(0, D)] = s * jnp.float32(scale)

  pool(1, si_o, L_OVRV, wv_o, D_OVRV, 1.0 / L_OVRV)
  pool(3, si_c, L_CAST, wv_c, D_CAST, 1.0 / L_CAST)
  pool(4, si_g, L_GENRE, wv_g, D_GENRE, 1.0 / L_GENRE)
  pool(5, si_p, L_PC, wv_p, D_PC, 1.0 / L_PC)
  pool(6, si_q, L_PCO, wv_q, D_PCO, 1.0 / L_PCO)
  out_ref[pl.ds(7, 1), pl.ds(0, D_NUM)] = num_in[...]
  ht.wait()
  hd.wait()
  out_ref[pl.ds(0, 1), pl.ds(0, D_TITLE)] = rt[...]
  out_ref[pl.ds(2, 1), pl.ds(0, D_DIR)] = rd[...]


@jax.jit
def _tc_call(si_o, si_c, si_g, si_p, si_q, si_t, si_d, num_row, wt, wo, wd,
             wc, wg, wp, wq):
  out8 = pl.pallas_call(
      _body,
      out_shape=jax.ShapeDtypeStruct((8, 128), jnp.float32),
      in_specs=[pl.BlockSpec(memory_space=pltpu.SMEM)] * 7 +
               [pl.BlockSpec(memory_space=pltpu.VMEM)] +
               [pl.BlockSpec(memory_space=pltpu.MemorySpace.HBM)] * 7,
      out_specs=pl.BlockSpec(memory_space=pltpu.VMEM),
      scratch_shapes=[
          pltpu.VMEM((30000, D_OVRV), jnp.float32),
          pltpu.VMEM((20000, D_CAST), jnp.float32),
          pltpu.VMEM((32, D_GENRE), jnp.float32),
          pltpu.VMEM((10000, D_PC), jnp.float32),
          pltpu.VMEM((200, D_PCO), jnp.float32),
          pltpu.VMEM((1, D_TITLE), jnp.float32),
          pltpu.VMEM((1, D_DIR), jnp.float32),
          pltpu.SemaphoreType.DMA((2,)),
          pltpu.SemaphoreType.DMA((5,)),
      ],
  )(si_o, si_c, si_g, si_p, si_q, si_t, si_d, num_row, wt, wo, wd, wc, wg,
    wp, wq)
  return jnp.concatenate(
      (out8[0, :D_TITLE], out8[1, :D_OVRV], out8[2, :D_DIR],
       out8[3, :D_CAST], out8[4, :D_GENRE], out8[5, :D_PC], out8[6, :D_PCO],
       out8[7, :D_NUM]))


def kernel(movie_ids, title, overrview, director, cast, genre,
           production_compaines, production_countries, numeric_movie_data,
           W_title, W_ovrv, W_dir, W_cast, W_genre, W_pc, W_pco):
  m = jnp.asarray(movie_ids, jnp.int32) - 1
  sl = lambda a: jax.lax.dynamic_slice_in_dim(a, m, 1, 0)
  t_row = jnp.reshape(jax.lax.dynamic_slice_in_dim(title, m, 1, 0), (1, 1))
  d_row = jnp.reshape(jax.lax.dynamic_slice_in_dim(director, m, 1, 0), (1, 1))
  return _tc_call(sl(overrview), sl(cast), sl(genre),
                  sl(production_compaines), sl(production_countries),
                  t_row, d_row, sl(numeric_movie_data), W_title, W_ovrv,
                  W_dir, W_cast, W_genre, W_pc, W_pco)
